# baseline (speedup 1.0000x reference)
import functools

import jax
import jax.numpy as jnp
from jax import lax
from jax.experimental import pallas as pl
from jax.experimental.pallas import tpu as pltpu
from jax.experimental.pallas import tpu_sc as plsc

S = 2048
D = 1024
H = 16
DH = 64
F = 4096
RB = 256
QB = 256
HG = 4
NEG = -5e4

f32 = jnp.float32
bf16 = jnp.bfloat16
i32 = jnp.int32
HI = lax.Precision.HIGHEST
DNT = (((1,), (1,)), ((), ()))
DNN = (((1,), (0,)), ((), ()))


def _rne_bf16(x):
    b = lax.bitcast_convert_type(x, i32)
    r = (b + 0x7FFF + ((b >> 16) & 1)) & jnp.int32(-65536)
    return lax.bitcast_convert_type(r, f32)


def _split(x):
    hi32 = _rne_bf16(x)
    hi = hi32.astype(bf16)
    lo = _rne_bf16(x - hi32).astype(bf16)
    return hi, lo


def _mm3(a, bh, bl, dn=DNN):
    ah, al = _split(a)
    d = lambda x, y: lax.dot_general(x, y, dn, preferred_element_type=f32)
    return d(ah, bh) + d(ah, bl) + d(al, bh)


def _gather_sc(table, ids):
    NW = 32
    BPW = S // NW
    mesh = plsc.VectorSubcoreMesh(core_axis_name="c", subcore_axis_name="s")

    @functools.partial(
        pl.kernel, mesh=mesh,
        out_type=jax.ShapeDtypeStruct((S, D), table.dtype),
        scratch_types=[
            pltpu.VMEM((BPW,), jnp.int32),
            pltpu.VMEM((BPW, D), table.dtype),
            pltpu.SemaphoreType.DMA,
        ],
    )
    def k(table_hbm, idx_hbm, out_hbm, idx_v, rows_v, sem):
        wid = lax.axis_index("s") * 2 + lax.axis_index("c")
        base = wid * BPW
        pltpu.sync_copy(idx_hbm.at[pl.ds(base, BPW)], idx_v)
        pltpu.async_copy(table_hbm.at[idx_v], rows_v, sem).wait()
        pltpu.sync_copy(rows_v, out_hbm.at[pl.ds(base, BPW)])

    return k(table, ids)


def _seg_masks():
    a = lax.broadcasted_iota(jnp.int32, (D, H), 0) // DH
    b = lax.broadcasted_iota(jnp.int32, (D, H), 1)
    seg = (a == b).astype(f32)
    c = lax.broadcasted_iota(jnp.int32, (H, D), 0)
    d = lax.broadcasted_iota(jnp.int32, (H, D), 1) // DH
    segT = (c == d).astype(f32)
    return seg, segT


def _ln(x, g, b):
    m = jnp.mean(x, axis=1, keepdims=True)
    xc = x - m
    var = jnp.mean(xc * xc, axis=1, keepdims=True)
    return xc * lax.rsqrt(var + 1e-5) * g + b


def _ln_qkv_common(x, g_ref, b_ref, wqkh_ref, wqkl_ref, wvh_ref, wvl_ref,
                   qk_ref, knh_ref, knl_ref, vh_ref, vl_ref):
    a = _ln(x, g_ref[...], b_ref[...])
    qk = _mm3(a, wqkh_ref[...], wqkl_ref[...])
    v = _mm3(a, wvh_ref[...], wvl_ref[...])
    qk_ref[...] = qk
    vh, vl = _split(v)
    vh_ref[...] = vh
    vl_ref[...] = vl
    seg, segT = _seg_masks()
    s2 = jnp.dot(qk * qk, seg, preferred_element_type=f32,
                 precision=HI)
    inv = 1.0 / jnp.maximum(jnp.sqrt(s2), 1e-12)
    bc = jnp.dot(inv, segT, preferred_element_type=f32,
                 precision=HI)
    knh, knl = _split(qk * bc)
    knh_ref[...] = knh
    knl_ref[...] = knl


def _ln_qkv_body_pos(x_ref, pos_ref, g_ref, b_ref, wqkh_ref, wqkl_ref,
                     wvh_ref, wvl_ref, qk_ref, knh_ref, knl_ref,
                     vh_ref, vl_ref, xp_ref):
    x = x_ref[...] + pos_ref[...]
    xp_ref[...] = x
    _ln_qkv_common(x, g_ref, b_ref, wqkh_ref, wqkl_ref, wvh_ref, wvl_ref,
                   qk_ref, knh_ref, knl_ref, vh_ref, vl_ref)


def _ln_qkv_body(x_ref, g_ref, b_ref, wqkh_ref, wqkl_ref, wvh_ref, wvl_ref,
                 qk_ref, knh_ref, knl_ref, vh_ref, vl_ref):
    _ln_qkv_common(x_ref[...], g_ref, b_ref, wqkh_ref, wqkl_ref,
                   wvh_ref, wvl_ref, qk_ref, knh_ref, knl_ref,
                   vh_ref, vl_ref)


def _ln_qkv(x, pos, g, b, wqkh, wqkl, wvh, wvl):
    nblk = S // RB
    row = pl.BlockSpec((RB, D), lambda i: (i, 0))
    cst = pl.BlockSpec((1, D), lambda i: (0, 0))
    wsp = pl.BlockSpec((D, D), lambda i: (0, 0))
    outs = [jax.ShapeDtypeStruct((S, D), f32)] + \
        [jax.ShapeDtypeStruct((S, D), bf16)] * 4
    if pos is not None:
        return pl.pallas_call(
            _ln_qkv_body_pos,
            grid=(nblk,),
            in_specs=[row, row, cst, cst, wsp, wsp, wsp, wsp],
            out_specs=[row] * 6,
            out_shape=outs + [jax.ShapeDtypeStruct((S, D), f32)],
        )(x, pos, g, b, wqkh, wqkl, wvh, wvl)
    return pl.pallas_call(
        _ln_qkv_body,
        grid=(nblk,),
        in_specs=[row, cst, cst, wsp, wsp, wsp, wsp],
        out_specs=[row] * 5,
        out_shape=outs,
    )(x, g, b, wqkh, wqkl, wvh, wvl)


def _attn_body(q_ref, knh_ref, knl_ref, vh_ref, vl_ref, o_ref):
    qb = pl.program_id(1)
    q = q_ref[...]
    knh = knh_ref[...]
    knl = knl_ref[...]
    vh = vh_ref[...]
    vl = vl_ref[...]
    lane_h = lax.broadcasted_iota(jnp.int32, (QB, HG * DH), 1) // DH
    row_g = qb * QB + lax.broadcasted_iota(jnp.int32, (QB, S), 0)
    col = lax.broadcasted_iota(jnp.int32, (QB, S), 1)
    selfm = col == row_g
    dd = lambda x, y: lax.dot_general(x, y, DNT, preferred_element_type=f32)
    acc = jnp.zeros((QB, HG * DH), f32)
    for hh in range(HG):
        qm = jnp.where(lane_h == hh, q, 0.0)
        qmh, qml = _split(qm)
        d = (dd(qmh, knh) + dd(qmh, knl) + dd(qml, knh)) * (DH ** -0.5)
        d = jnp.where(selfm, NEG, d)
        e = jnp.exp(d)
        rs = 1.0 / jnp.sum(e, axis=1, keepdims=True)
        eh, el = _split(e)
        t = (jnp.dot(eh, vh, preferred_element_type=f32)
             + jnp.dot(eh, vl, preferred_element_type=f32)
             + jnp.dot(el, vh, preferred_element_type=f32)) * rs
        acc = acc + jnp.where(lane_h == hh, t, 0.0)
    o_ref[...] = acc


def _attn_full(qk, knh, knl, vh, vl):
    hw = HG * DH
    qspec = pl.BlockSpec((QB, hw), lambda h, q: (q, h))
    kspec = pl.BlockSpec((S, hw), lambda h, q: (0, h))
    return pl.pallas_call(
        _attn_body,
        grid=(H // HG, S // QB),
        in_specs=[qspec, kspec, kspec, kspec, kspec],
        out_specs=qspec,
        out_shape=jax.ShapeDtypeStruct((S, D), f32),
    )(qk, knh, knl, vh, vl)


def _oproj_body(o_ref, x1_ref, woh_ref, wol_ref, bo_ref, g_ref, b_ref,
                y1_ref, a2_ref):
    y1 = x1_ref[...] + _mm3(o_ref[...], woh_ref[...], wol_ref[...]) \
        + bo_ref[...]
    y1_ref[...] = y1
    a2_ref[...] = _ln(y1, g_ref[...], b_ref[...])


def _oproj_ln(o, x1, woh, wol, bo, g, b):
    row = pl.BlockSpec((RB, D), lambda i: (i, 0))
    cst = pl.BlockSpec((1, D), lambda i: (0, 0))
    wsp = pl.BlockSpec((D, D), lambda i: (0, 0))
    return pl.pallas_call(
        _oproj_body,
        grid=(S // RB,),
        in_specs=[row, row, wsp, wsp, cst, cst, cst],
        out_specs=[row, row],
        out_shape=[jax.ShapeDtypeStruct((S, D), f32)] * 2,
    )(o, x1, woh, wol, bo, g, b)


def _gelu(x):
    return x * 0.5 * (1.0 + lax.erf(x * (2.0 ** -0.5)))


def _ffn_body(a_ref, x2_ref, w1h_ref, w1l_ref, b1_ref, w2h_ref, w2l_ref,
              b2_ref, y2_ref):
    hdd = _gelu(_mm3(a_ref[...], w1h_ref[...], w1l_ref[...]) + b1_ref[...])
    y2_ref[...] = x2_ref[...] + _mm3(hdd, w2h_ref[...], w2l_ref[...]) \
        + b2_ref[...]


def _ffn(a, x2, w1h, w1l, b1, w2h, w2l, b2):
    row = pl.BlockSpec((RB, D), lambda i: (i, 0))
    return pl.pallas_call(
        _ffn_body,
        grid=(S // RB,),
        in_specs=[
            row, row,
            pl.BlockSpec((D, F), lambda i: (0, 0)),
            pl.BlockSpec((D, F), lambda i: (0, 0)),
            pl.BlockSpec((1, F), lambda i: (0, 0)),
            pl.BlockSpec((F, D), lambda i: (0, 0)),
            pl.BlockSpec((F, D), lambda i: (0, 0)),
            pl.BlockSpec((1, D), lambda i: (0, 0)),
        ],
        out_specs=row,
        out_shape=jax.ShapeDtypeStruct((S, D), f32),
    )(a, x2, w1h, w1l, b1, w2h, w2l, b2)


def _attn0_body(q_ref, knh_ref, knl_ref, vh_ref, vl_ref, o_ref):
    q = q_ref[...]
    Q = jnp.concatenate([q] * H, axis=0)
    lane_h = lax.broadcasted_iota(jnp.int32, (H * 8, D), 1) // DH
    row_h = lax.broadcasted_iota(jnp.int32, (H * 8, D), 0) // 8
    hm = lane_h == row_h
    Qm = jnp.where(hm, Q, 0.0)
    Qmh, Qml = _split(Qm)
    dd = lambda x, y: lax.dot_general(x, y, DNT, preferred_element_type=f32)
    knh = knh_ref[...]
    knl = knl_ref[...]
    d = (dd(Qmh, knh) + dd(Qmh, knl) + dd(Qml, knh)) * (DH ** -0.5)
    col = lax.broadcasted_iota(jnp.int32, (H * 8, S), 1)
    r = lax.broadcasted_iota(jnp.int32, (H * 8, S), 0) % 8
    d = jnp.where(col == r, NEG, d)
    e = jnp.exp(d)
    rs = 1.0 / jnp.sum(e, axis=1, keepdims=True)
    eh, el = _split(e)
    vh = vh_ref[...]
    vl = vl_ref[...]
    t = (jnp.dot(eh, vh, preferred_element_type=f32)
         + jnp.dot(eh, vl, preferred_element_type=f32)
         + jnp.dot(el, vh, preferred_element_type=f32)) * rs
    tm = jnp.where(hm, t, 0.0)
    o = jnp.zeros((8, D), f32)
    for hh in range(H):
        o = o + tm[hh * 8:(hh + 1) * 8, :]
    o_ref[...] = o


def _attn_row0(qk, knh, knl, vh, vl):
    full = pl.BlockSpec((S, D), lambda i: (0, 0))
    return pl.pallas_call(
        _attn0_body,
        grid=(1,),
        in_specs=[pl.BlockSpec((8, D), lambda i: (0, 0)),
                  full, full, full, full],
        out_specs=pl.BlockSpec((8, D), lambda i: (0, 0)),
        out_shape=jax.ShapeDtypeStruct((8, D), f32),
    )(qk, knh, knl, vh, vl)


def _tail_body(o_ref, y1r_ref, y2r_ref, woh_ref, wol_ref, bo_ref,
               g_ref, b_ref, w1h_ref, w1l_ref, b1_ref, w2h_ref, w2l_ref,
               b2_ref, wc_ref, bc_ref, out_ref):
    y1f = y1r_ref[...] + _mm3(o_ref[...], woh_ref[...], wol_ref[...]) \
        + bo_ref[...]
    a = _ln(y1f, g_ref[...], b_ref[...])
    hdd = _gelu(_mm3(a, w1h_ref[...], w1l_ref[...]) + b1_ref[...])
    y2f = y2r_ref[...] + _mm3(hdd, w2h_ref[...], w2l_ref[...]) + b2_ref[...]
    hsum = y1f + y2f
    out_ref[...] = jnp.dot(hsum, wc_ref[...], preferred_element_type=f32,
                           precision=HI) + bc_ref[...]


def _tail(o8, y1r, y2r, woh, wol, bo, g, b, w1h, w1l, b1, w2h, w2l, b2,
          wc, bc):
    full = lambda shape: pl.BlockSpec(shape, lambda i: tuple(0 for _ in shape))
    return pl.pallas_call(
        _tail_body,
        grid=(1,),
        in_specs=[
            full((8, D)), full((8, D)), full((8, D)),
            full((D, D)), full((D, D)), full((1, D)), full((1, D)),
            full((1, D)),
            full((D, F)), full((D, F)), full((1, F)),
            full((F, D)), full((F, D)), full((1, D)),
            full((D, 2)), full((1, 2)),
        ],
        out_specs=full((8, 2)),
        out_shape=jax.ShapeDtypeStruct((8, 2), f32),
    )(o8, y1r, y2r, woh, wol, bo, g, b, w1h, w1l, b1, w2h, w2l, b2, wc, bc)


def _wsplit(w):
    hi32 = _rne_bf16(w)
    hi = hi32.astype(bf16)
    lo = _rne_bf16(w - hi32).astype(bf16)
    return hi, lo


def kernel(input_ids, token_emb, pos_emb, ln1_g, ln1_b, Wqk, Wv, Wo, bo,
           ln2_g, ln2_b, W1, b1, W2, b2, Wc, bc):
    ids = input_ids.reshape(S).astype(jnp.int32)
    emb = _gather_sc(token_emb, ids)

    r1 = lambda a: a.reshape(1, -1)
    wqk = [_wsplit(Wqk[i]) for i in range(2)]
    wv = [_wsplit(Wv[i]) for i in range(2)]
    wo = [_wsplit(Wo[i]) for i in range(2)]
    w1 = [_wsplit(W1[i]) for i in range(2)]
    w2 = [_wsplit(W2[i]) for i in range(2)]

    qk0, knh0, knl0, vh0, vl0, x0 = _ln_qkv(
        emb, pos_emb, r1(ln1_g[0]), r1(ln1_b[0]), *wqk[0], *wv[0])
    o0 = _attn_full(qk0, knh0, knl0, vh0, vl0)
    y1, a2 = _oproj_ln(o0, x0, *wo[0], r1(bo[0]), r1(ln2_g[0]),
                       r1(ln2_b[0]))
    y2 = _ffn(a2, x0, *w1[0], r1(b1[0]), *w2[0], r1(b2[0]))

    qk1, knh1, knl1, vh1, vl1 = _ln_qkv(y2, None, r1(ln1_g[1]),
                                        r1(ln1_b[1]), *wqk[1], *wv[1])
    o8 = _attn_row0(qk1, knh1, knl1, vh1, vl1)
    out8 = _tail(o8, lax.slice(y1, (0, 0), (8, D)),
                 lax.slice(y2, (0, 0), (8, D)),
                 *wo[1], r1(bo[1]), r1(ln2_g[1]), r1(ln2_b[1]),
                 *w1[1], r1(b1[1]), *w2[1], r1(b2[1]), Wc, r1(bc))
    return out8[0:1, :]


# speedup vs baseline: 1.3552x; 1.3552x over previous
import functools

import jax
import jax.numpy as jnp
from jax import lax
from jax.experimental import pallas as pl
from jax.experimental.pallas import tpu as pltpu
from jax.experimental.pallas import tpu_sc as plsc

S = 2048
D = 1024
H = 16
DH = 64
F = 4096
RB = 256
QB = 256
HG = 4
NEG = -5e4

f32 = jnp.float32
bf16 = jnp.bfloat16
i32 = jnp.int32
HI = lax.Precision.HIGHEST
DNT = (((1,), (1,)), ((), ()))
DNN = (((1,), (0,)), ((), ()))


def _rne_bf16(x):
    b = lax.bitcast_convert_type(x, i32)
    r = (b + 0x7FFF + ((b >> 16) & 1)) & jnp.int32(-65536)
    return lax.bitcast_convert_type(r, f32)


def _split(x):
    hi32 = _rne_bf16(x)
    hi = hi32.astype(bf16)
    lo = _rne_bf16(x - hi32).astype(bf16)
    return hi, lo


def _mm3(a, bh, bl, dn=DNN):
    ah, al = _split(a)
    d = lambda x, y: lax.dot_general(x, y, dn, preferred_element_type=f32)
    return d(ah, bh) + d(ah, bl) + d(al, bh)


def _gather_sc(table, ids):
    NW = 32
    BPW = S // NW
    mesh = plsc.VectorSubcoreMesh(core_axis_name="c", subcore_axis_name="s")

    @functools.partial(
        pl.kernel, mesh=mesh,
        out_type=jax.ShapeDtypeStruct((S, D), table.dtype),
        scratch_types=[
            pltpu.VMEM((BPW,), jnp.int32),
            pltpu.VMEM((BPW, D), table.dtype),
            pltpu.SemaphoreType.DMA,
        ],
    )
    def k(table_hbm, idx_hbm, out_hbm, idx_v, rows_v, sem):
        wid = lax.axis_index("s") * 2 + lax.axis_index("c")
        base = wid * BPW
        pltpu.sync_copy(idx_hbm.at[pl.ds(base, BPW)], idx_v)
        pltpu.async_copy(table_hbm.at[idx_v], rows_v, sem).wait()
        pltpu.sync_copy(rows_v, out_hbm.at[pl.ds(base, BPW)])

    return k(table, ids)


def _seg_masks():
    a = lax.broadcasted_iota(jnp.int32, (D, H), 0) // DH
    b = lax.broadcasted_iota(jnp.int32, (D, H), 1)
    seg = (a == b).astype(f32)
    c = lax.broadcasted_iota(jnp.int32, (H, D), 0)
    d = lax.broadcasted_iota(jnp.int32, (H, D), 1) // DH
    segT = (c == d).astype(f32)
    return seg, segT


def _ln(x, g, b):
    m = jnp.mean(x, axis=1, keepdims=True)
    xc = x - m
    var = jnp.mean(xc * xc, axis=1, keepdims=True)
    return xc * lax.rsqrt(var + 1e-5) * g + b


def _ln_qkv_common(x, g_ref, b_ref, wqkh_ref, wqkl_ref, wvh_ref, wvl_ref,
                   qk_ref, knh_ref, knl_ref, vh_ref, vl_ref):
    a = _ln(x, g_ref[...], b_ref[...])
    qk = _mm3(a, wqkh_ref[...], wqkl_ref[...])
    v = _mm3(a, wvh_ref[...], wvl_ref[...])
    qk_ref[...] = qk
    vh, vl = _split(v)
    vh_ref[...] = vh
    vl_ref[...] = vl
    seg, segT = _seg_masks()
    s2 = jnp.dot(qk * qk, seg, preferred_element_type=f32,
                 precision=HI)
    inv = 1.0 / jnp.maximum(jnp.sqrt(s2), 1e-12)
    bc = jnp.dot(inv, segT, preferred_element_type=f32,
                 precision=HI)
    knh, knl = _split(qk * bc)
    knh_ref[...] = knh
    knl_ref[...] = knl


def _ln_qkv_body_pos(x_ref, pos_ref, g_ref, b_ref, wqkh_ref, wqkl_ref,
                     wvh_ref, wvl_ref, qk_ref, knh_ref, knl_ref,
                     vh_ref, vl_ref, xp_ref):
    x = x_ref[...] + pos_ref[...]
    xp_ref[...] = x
    _ln_qkv_common(x, g_ref, b_ref, wqkh_ref, wqkl_ref, wvh_ref, wvl_ref,
                   qk_ref, knh_ref, knl_ref, vh_ref, vl_ref)


def _ln_qkv_body(x_ref, g_ref, b_ref, wqkh_ref, wqkl_ref, wvh_ref, wvl_ref,
                 qk_ref, knh_ref, knl_ref, vh_ref, vl_ref):
    _ln_qkv_common(x_ref[...], g_ref, b_ref, wqkh_ref, wqkl_ref,
                   wvh_ref, wvl_ref, qk_ref, knh_ref, knl_ref,
                   vh_ref, vl_ref)


def _ln_qkv(x, pos, g, b, wqkh, wqkl, wvh, wvl):
    nblk = S // RB
    row = pl.BlockSpec((RB, D), lambda i: (i, 0))
    cst = pl.BlockSpec((1, D), lambda i: (0, 0))
    wsp = pl.BlockSpec((D, D), lambda i: (0, 0))
    outs = [jax.ShapeDtypeStruct((S, D), f32)] + \
        [jax.ShapeDtypeStruct((S, D), bf16)] * 4
    if pos is not None:
        return pl.pallas_call(
            _ln_qkv_body_pos,
            grid=(nblk,),
            in_specs=[row, row, cst, cst, wsp, wsp, wsp, wsp],
            out_specs=[row] * 6,
            out_shape=outs + [jax.ShapeDtypeStruct((S, D), f32)],
        )(x, pos, g, b, wqkh, wqkl, wvh, wvl)
    return pl.pallas_call(
        _ln_qkv_body,
        grid=(nblk,),
        in_specs=[row, cst, cst, wsp, wsp, wsp, wsp],
        out_specs=[row] * 5,
        out_shape=outs,
    )(x, g, b, wqkh, wqkl, wvh, wvl)


def _attn_body(q_ref, knh_ref, knl_ref, vh_ref, vl_ref, o_ref):
    qb = pl.program_id(1)
    q = q_ref[...]
    knh = knh_ref[...]
    knl = knl_ref[...]
    vh = vh_ref[...]
    vl = vl_ref[...]
    lane_h = lax.broadcasted_iota(jnp.int32, (QB, HG * DH), 1) // DH
    row_g = qb * QB + lax.broadcasted_iota(jnp.int32, (QB, S), 0)
    col = lax.broadcasted_iota(jnp.int32, (QB, S), 1)
    selfm = col == row_g
    dd = lambda x, y: lax.dot_general(x, y, DNT, preferred_element_type=f32)
    acc = jnp.zeros((QB, HG * DH), f32)
    for hh in range(HG):
        qm = jnp.where(lane_h == hh, q, 0.0)
        qmh = _rne_bf16(qm).astype(bf16)
        d = dd(qmh, knh) * (DH ** -0.5)
        d = jnp.where(selfm, NEG, d)
        e = jnp.exp(d)
        rs = 1.0 / jnp.sum(e, axis=1, keepdims=True)
        eh = _rne_bf16(e).astype(bf16)
        t = jnp.dot(eh, vh, preferred_element_type=f32) * rs
        acc = acc + jnp.where(lane_h == hh, t, 0.0)
    o_ref[...] = acc


def _attn_full(qk, knh, knl, vh, vl):
    hw = HG * DH
    qspec = pl.BlockSpec((QB, hw), lambda h, q: (q, h))
    kspec = pl.BlockSpec((S, hw), lambda h, q: (0, h))
    return pl.pallas_call(
        _attn_body,
        grid=(H // HG, S // QB),
        in_specs=[qspec, kspec, kspec, kspec, kspec],
        out_specs=qspec,
        out_shape=jax.ShapeDtypeStruct((S, D), f32),
    )(qk, knh, knl, vh, vl)


def _oproj_body(o_ref, x1_ref, woh_ref, wol_ref, bo_ref, g_ref, b_ref,
                y1_ref, a2_ref):
    y1 = x1_ref[...] + _mm3(o_ref[...], woh_ref[...], wol_ref[...]) \
        + bo_ref[...]
    y1_ref[...] = y1
    a2_ref[...] = _ln(y1, g_ref[...], b_ref[...])


def _oproj_ln(o, x1, woh, wol, bo, g, b):
    row = pl.BlockSpec((RB, D), lambda i: (i, 0))
    cst = pl.BlockSpec((1, D), lambda i: (0, 0))
    wsp = pl.BlockSpec((D, D), lambda i: (0, 0))
    return pl.pallas_call(
        _oproj_body,
        grid=(S // RB,),
        in_specs=[row, row, wsp, wsp, cst, cst, cst],
        out_specs=[row, row],
        out_shape=[jax.ShapeDtypeStruct((S, D), f32)] * 2,
    )(o, x1, woh, wol, bo, g, b)


def _gelu(x):
    return x * 0.5 * (1.0 + lax.erf(x * (2.0 ** -0.5)))


def _ffn_body(a_ref, x2_ref, w1h_ref, w1l_ref, b1_ref, w2h_ref, w2l_ref,
              b2_ref, y2_ref):
    hdd = _gelu(_mm3(a_ref[...], w1h_ref[...], w1l_ref[...]) + b1_ref[...])
    y2_ref[...] = x2_ref[...] + _mm3(hdd, w2h_ref[...], w2l_ref[...]) \
        + b2_ref[...]


def _ffn(a, x2, w1h, w1l, b1, w2h, w2l, b2):
    row = pl.BlockSpec((RB, D), lambda i: (i, 0))
    return pl.pallas_call(
        _ffn_body,
        grid=(S // RB,),
        in_specs=[
            row, row,
            pl.BlockSpec((D, F), lambda i: (0, 0)),
            pl.BlockSpec((D, F), lambda i: (0, 0)),
            pl.BlockSpec((1, F), lambda i: (0, 0)),
            pl.BlockSpec((F, D), lambda i: (0, 0)),
            pl.BlockSpec((F, D), lambda i: (0, 0)),
            pl.BlockSpec((1, D), lambda i: (0, 0)),
        ],
        out_specs=row,
        out_shape=jax.ShapeDtypeStruct((S, D), f32),
    )(a, x2, w1h, w1l, b1, w2h, w2l, b2)


def _attn0_body(q_ref, knh_ref, knl_ref, vh_ref, vl_ref, o_ref):
    q = q_ref[...]
    Q = jnp.concatenate([q] * H, axis=0)
    lane_h = lax.broadcasted_iota(jnp.int32, (H * 8, D), 1) // DH
    row_h = lax.broadcasted_iota(jnp.int32, (H * 8, D), 0) // 8
    hm = lane_h == row_h
    Qm = jnp.where(hm, Q, 0.0)
    Qmh, Qml = _split(Qm)
    dd = lambda x, y: lax.dot_general(x, y, DNT, preferred_element_type=f32)
    knh = knh_ref[...]
    knl = knl_ref[...]
    d = (dd(Qmh, knh) + dd(Qmh, knl) + dd(Qml, knh)) * (DH ** -0.5)
    col = lax.broadcasted_iota(jnp.int32, (H * 8, S), 1)
    r = lax.broadcasted_iota(jnp.int32, (H * 8, S), 0) % 8
    d = jnp.where(col == r, NEG, d)
    e = jnp.exp(d)
    rs = 1.0 / jnp.sum(e, axis=1, keepdims=True)
    eh, el = _split(e)
    vh = vh_ref[...]
    vl = vl_ref[...]
    t = (jnp.dot(eh, vh, preferred_element_type=f32)
         + jnp.dot(eh, vl, preferred_element_type=f32)
         + jnp.dot(el, vh, preferred_element_type=f32)) * rs
    tm = jnp.where(hm, t, 0.0)
    o = jnp.zeros((8, D), f32)
    for hh in range(H):
        o = o + tm[hh * 8:(hh + 1) * 8, :]
    o_ref[...] = o


def _attn_row0(qk, knh, knl, vh, vl):
    full = pl.BlockSpec((S, D), lambda i: (0, 0))
    return pl.pallas_call(
        _attn0_body,
        grid=(1,),
        in_specs=[pl.BlockSpec((8, D), lambda i: (0, 0)),
                  full, full, full, full],
        out_specs=pl.BlockSpec((8, D), lambda i: (0, 0)),
        out_shape=jax.ShapeDtypeStruct((8, D), f32),
    )(qk, knh, knl, vh, vl)


def _tail_body(o_ref, y1r_ref, y2r_ref, woh_ref, wol_ref, bo_ref,
               g_ref, b_ref, w1h_ref, w1l_ref, b1_ref, w2h_ref, w2l_ref,
               b2_ref, wc_ref, bc_ref, out_ref):
    y1f = y1r_ref[...] + _mm3(o_ref[...], woh_ref[...], wol_ref[...]) \
        + bo_ref[...]
    a = _ln(y1f, g_ref[...], b_ref[...])
    hdd = _gelu(_mm3(a, w1h_ref[...], w1l_ref[...]) + b1_ref[...])
    y2f = y2r_ref[...] + _mm3(hdd, w2h_ref[...], w2l_ref[...]) + b2_ref[...]
    hsum = y1f + y2f
    out_ref[...] = jnp.dot(hsum, wc_ref[...], preferred_element_type=f32,
                           precision=HI) + bc_ref[...]


def _tail(o8, y1r, y2r, woh, wol, bo, g, b, w1h, w1l, b1, w2h, w2l, b2,
          wc, bc):
    full = lambda shape: pl.BlockSpec(shape, lambda i: tuple(0 for _ in shape))
    return pl.pallas_call(
        _tail_body,
        grid=(1,),
        in_specs=[
            full((8, D)), full((8, D)), full((8, D)),
            full((D, D)), full((D, D)), full((1, D)), full((1, D)),
            full((1, D)),
            full((D, F)), full((D, F)), full((1, F)),
            full((F, D)), full((F, D)), full((1, D)),
            full((D, 2)), full((1, 2)),
        ],
        out_specs=full((8, 2)),
        out_shape=jax.ShapeDtypeStruct((8, 2), f32),
    )(o8, y1r, y2r, woh, wol, bo, g, b, w1h, w1l, b1, w2h, w2l, b2, wc, bc)


def _wsplit(w):
    hi32 = _rne_bf16(w)
    hi = hi32.astype(bf16)
    lo = _rne_bf16(w - hi32).astype(bf16)
    return hi, lo


def kernel(input_ids, token_emb, pos_emb, ln1_g, ln1_b, Wqk, Wv, Wo, bo,
           ln2_g, ln2_b, W1, b1, W2, b2, Wc, bc):
    ids = input_ids.reshape(S).astype(jnp.int32)
    emb = _gather_sc(token_emb, ids)

    r1 = lambda a: a.reshape(1, -1)
    wqk = [_wsplit(Wqk[i]) for i in range(2)]
    wv = [_wsplit(Wv[i]) for i in range(2)]
    wo = [_wsplit(Wo[i]) for i in range(2)]
    w1 = [_wsplit(W1[i]) for i in range(2)]
    w2 = [_wsplit(W2[i]) for i in range(2)]

    qk0, knh0, knl0, vh0, vl0, x0 = _ln_qkv(
        emb, pos_emb, r1(ln1_g[0]), r1(ln1_b[0]), *wqk[0], *wv[0])
    o0 = _attn_full(qk0, knh0, knl0, vh0, vl0)
    y1, a2 = _oproj_ln(o0, x0, *wo[0], r1(bo[0]), r1(ln2_g[0]),
                       r1(ln2_b[0]))
    y2 = _ffn(a2, x0, *w1[0], r1(b1[0]), *w2[0], r1(b2[0]))

    qk1, knh1, knl1, vh1, vl1 = _ln_qkv(y2, None, r1(ln1_g[1]),
                                        r1(ln1_b[1]), *wqk[1], *wv[1])
    o8 = _attn_row0(qk1, knh1, knl1, vh1, vl1)
    out8 = _tail(o8, lax.slice(y1, (0, 0), (8, D)),
                 lax.slice(y2, (0, 0), (8, D)),
                 *wo[1], r1(bo[1]), r1(ln2_g[1]), r1(ln2_b[1]),
                 *w1[1], r1(b1[1]), *w2[1], r1(b2[1]), Wc, r1(bc))
    return out8[0:1, :]


# speedup vs baseline: 1.4962x; 1.1041x over previous
import functools

import jax
import jax.numpy as jnp
from jax import lax
from jax.experimental import pallas as pl
from jax.experimental.pallas import tpu as pltpu
from jax.experimental.pallas import tpu_sc as plsc

S = 2048
D = 1024
H = 16
DH = 64
F = 4096
RB = 256
QB = 256
HG = 4
NEG = -5e4

f32 = jnp.float32
bf16 = jnp.bfloat16
i32 = jnp.int32
HI = lax.Precision.HIGHEST
DNT = (((1,), (1,)), ((), ()))
DNN = (((1,), (0,)), ((), ()))


def _rne_bf16(x):
    b = lax.bitcast_convert_type(x, i32)
    r = (b + 0x7FFF + ((b >> 16) & 1)) & jnp.int32(-65536)
    return lax.bitcast_convert_type(r, f32)


def _split(x):
    hi = x.astype(bf16)
    lo = (x - hi.astype(f32)).astype(bf16)
    return hi, lo


def _mm3(a, bh, bl, dn=DNN):
    ah, al = _split(a)
    d = lambda x, y: lax.dot_general(x, y, dn, preferred_element_type=f32)
    return d(ah, bh) + d(ah, bl) + d(al, bh)


def _gather_sc(table, ids):
    NW = 32
    BPW = S // NW
    mesh = plsc.VectorSubcoreMesh(core_axis_name="c", subcore_axis_name="s")

    @functools.partial(
        pl.kernel, mesh=mesh,
        out_type=jax.ShapeDtypeStruct((S, D), table.dtype),
        scratch_types=[
            pltpu.VMEM((BPW,), jnp.int32),
            pltpu.VMEM((BPW, D), table.dtype),
            pltpu.SemaphoreType.DMA,
        ],
    )
    def k(table_hbm, idx_hbm, out_hbm, idx_v, rows_v, sem):
        wid = lax.axis_index("s") * 2 + lax.axis_index("c")
        base = wid * BPW
        pltpu.sync_copy(idx_hbm.at[pl.ds(base, BPW)], idx_v)
        pltpu.async_copy(table_hbm.at[idx_v], rows_v, sem).wait()
        pltpu.sync_copy(rows_v, out_hbm.at[pl.ds(base, BPW)])

    return k(table, ids)


def _seg_masks():
    a = lax.broadcasted_iota(jnp.int32, (D, H), 0) // DH
    b = lax.broadcasted_iota(jnp.int32, (D, H), 1)
    seg = (a == b).astype(f32)
    c = lax.broadcasted_iota(jnp.int32, (H, D), 0)
    d = lax.broadcasted_iota(jnp.int32, (H, D), 1) // DH
    segT = (c == d).astype(f32)
    return seg, segT


def _ln(x, g, b):
    m = jnp.mean(x, axis=1, keepdims=True)
    xc = x - m
    var = jnp.mean(xc * xc, axis=1, keepdims=True)
    return xc * lax.rsqrt(var + 1e-5) * g + b


def _ln_qkv_common(x, g_ref, b_ref, wqkh_ref, wqkl_ref, wvh_ref, wvl_ref,
                   qk_ref, knh_ref, vh_ref):
    a = _ln(x, g_ref[...], b_ref[...])
    qk = _mm3(a, wqkh_ref[...], wqkl_ref[...])
    v = _mm3(a, wvh_ref[...], wvl_ref[...])
    qk_ref[...] = qk
    vh_ref[...] = v.astype(bf16)
    seg, segT = _seg_masks()
    s2 = jnp.dot(qk * qk, seg, preferred_element_type=f32,
                 precision=HI)
    inv = 1.0 / jnp.maximum(jnp.sqrt(s2), 1e-12)
    bc = jnp.dot(inv, segT, preferred_element_type=f32,
                 precision=HI)
    knh_ref[...] = (qk * bc).astype(bf16)


def _ln_qkv_body_pos(x_ref, pos_ref, g_ref, b_ref, wqkh_ref, wqkl_ref,
                     wvh_ref, wvl_ref, qk_ref, knh_ref, vh_ref, xp_ref):
    x = x_ref[...] + pos_ref[...]
    xp_ref[...] = x
    _ln_qkv_common(x, g_ref, b_ref, wqkh_ref, wqkl_ref, wvh_ref, wvl_ref,
                   qk_ref, knh_ref, vh_ref)


def _ln_qkv_body(x_ref, g_ref, b_ref, wqkh_ref, wqkl_ref, wvh_ref, wvl_ref,
                 qk_ref, knh_ref, vh_ref):
    _ln_qkv_common(x_ref[...], g_ref, b_ref, wqkh_ref, wqkl_ref,
                   wvh_ref, wvl_ref, qk_ref, knh_ref, vh_ref)


def _ln_qkv(x, pos, g, b, wqkh, wqkl, wvh, wvl):
    nblk = S // RB
    row = pl.BlockSpec((RB, D), lambda i: (i, 0))
    cst = pl.BlockSpec((1, D), lambda i: (0, 0))
    wsp = pl.BlockSpec((D, D), lambda i: (0, 0))
    outs = [jax.ShapeDtypeStruct((S, D), f32)] + \
        [jax.ShapeDtypeStruct((S, D), bf16)] * 2
    if pos is not None:
        return pl.pallas_call(
            _ln_qkv_body_pos,
            grid=(nblk,),
            in_specs=[row, row, cst, cst, wsp, wsp, wsp, wsp],
            out_specs=[row] * 4,
            out_shape=outs + [jax.ShapeDtypeStruct((S, D), f32)],
        )(x, pos, g, b, wqkh, wqkl, wvh, wvl)
    return pl.pallas_call(
        _ln_qkv_body,
        grid=(nblk,),
        in_specs=[row, cst, cst, wsp, wsp, wsp, wsp],
        out_specs=[row] * 3,
        out_shape=outs,
    )(x, g, b, wqkh, wqkl, wvh, wvl)


def _attn_body(q_ref, knh_ref, vh_ref, o_ref):
    qb = pl.program_id(1)
    q = q_ref[...]
    knh = knh_ref[...]
    vh = vh_ref[...]
    lane_h = lax.broadcasted_iota(jnp.int32, (QB, HG * DH), 1) // DH
    row_g = qb * QB + lax.broadcasted_iota(jnp.int32, (QB, S), 0)
    col = lax.broadcasted_iota(jnp.int32, (QB, S), 1)
    selfm = col == row_g
    dd = lambda x, y: lax.dot_general(x, y, DNT, preferred_element_type=f32)
    acc = jnp.zeros((QB, HG * DH), f32)
    for hh in range(HG):
        qm = jnp.where(lane_h == hh, q, 0.0)
        qmh = qm.astype(bf16)
        d = dd(qmh, knh) * (DH ** -0.5)
        d = jnp.where(selfm, NEG, d)
        e = jnp.exp(d)
        rs = 1.0 / jnp.sum(e, axis=1, keepdims=True)
        eh = e.astype(bf16)
        t = jnp.dot(eh, vh, preferred_element_type=f32) * rs
        acc = acc + jnp.where(lane_h == hh, t, 0.0)
    o_ref[...] = acc


def _attn_full(qk, knh, vh):
    hw = HG * DH
    qspec = pl.BlockSpec((QB, hw), lambda h, q: (q, h))
    kspec = pl.BlockSpec((S, hw), lambda h, q: (0, h))
    return pl.pallas_call(
        _attn_body,
        grid=(H // HG, S // QB),
        in_specs=[qspec, kspec, kspec],
        out_specs=qspec,
        out_shape=jax.ShapeDtypeStruct((S, D), f32),
    )(qk, knh, vh)


def _oproj_body(o_ref, x1_ref, woh_ref, wol_ref, bo_ref, g_ref, b_ref,
                y1_ref, a2_ref):
    y1 = x1_ref[...] + _mm3(o_ref[...], woh_ref[...], wol_ref[...]) \
        + bo_ref[...]
    y1_ref[...] = y1
    a2_ref[...] = _ln(y1, g_ref[...], b_ref[...])


def _oproj_ln(o, x1, woh, wol, bo, g, b):
    row = pl.BlockSpec((RB, D), lambda i: (i, 0))
    cst = pl.BlockSpec((1, D), lambda i: (0, 0))
    wsp = pl.BlockSpec((D, D), lambda i: (0, 0))
    return pl.pallas_call(
        _oproj_body,
        grid=(S // RB,),
        in_specs=[row, row, wsp, wsp, cst, cst, cst],
        out_specs=[row, row],
        out_shape=[jax.ShapeDtypeStruct((S, D), f32)] * 2,
    )(o, x1, woh, wol, bo, g, b)


def _gelu(x):
    return x * 0.5 * (1.0 + lax.erf(x * (2.0 ** -0.5)))


def _ffn_body(a_ref, x2_ref, w1h_ref, w1l_ref, b1_ref, w2h_ref, w2l_ref,
              b2_ref, y2_ref):
    hdd = _gelu(_mm3(a_ref[...], w1h_ref[...], w1l_ref[...]) + b1_ref[...])
    y2_ref[...] = x2_ref[...] + _mm3(hdd, w2h_ref[...], w2l_ref[...]) \
        + b2_ref[...]


def _ffn(a, x2, w1h, w1l, b1, w2h, w2l, b2):
    row = pl.BlockSpec((RB, D), lambda i: (i, 0))
    return pl.pallas_call(
        _ffn_body,
        grid=(S // RB,),
        in_specs=[
            row, row,
            pl.BlockSpec((D, F), lambda i: (0, 0)),
            pl.BlockSpec((D, F), lambda i: (0, 0)),
            pl.BlockSpec((1, F), lambda i: (0, 0)),
            pl.BlockSpec((F, D), lambda i: (0, 0)),
            pl.BlockSpec((F, D), lambda i: (0, 0)),
            pl.BlockSpec((1, D), lambda i: (0, 0)),
        ],
        out_specs=row,
        out_shape=jax.ShapeDtypeStruct((S, D), f32),
    )(a, x2, w1h, w1l, b1, w2h, w2l, b2)


def _attn0_body(q_ref, knh_ref, vh_ref, o_ref):
    q = q_ref[...]
    Q = jnp.concatenate([q] * H, axis=0)
    lane_h = lax.broadcasted_iota(jnp.int32, (H * 8, D), 1) // DH
    row_h = lax.broadcasted_iota(jnp.int32, (H * 8, D), 0) // 8
    hm = lane_h == row_h
    Qm = jnp.where(hm, Q, 0.0)
    Qmh = Qm.astype(bf16)
    dd = lambda x, y: lax.dot_general(x, y, DNT, preferred_element_type=f32)
    d = dd(Qmh, knh_ref[...]) * (DH ** -0.5)
    col = lax.broadcasted_iota(jnp.int32, (H * 8, S), 1)
    r = lax.broadcasted_iota(jnp.int32, (H * 8, S), 0) % 8
    d = jnp.where(col == r, NEG, d)
    e = jnp.exp(d)
    rs = 1.0 / jnp.sum(e, axis=1, keepdims=True)
    eh = e.astype(bf16)
    t = jnp.dot(eh, vh_ref[...], preferred_element_type=f32) * rs
    tm = jnp.where(hm, t, 0.0)
    o = jnp.zeros((8, D), f32)
    for hh in range(H):
        o = o + tm[hh * 8:(hh + 1) * 8, :]
    o_ref[...] = o


def _attn_row0(qk, knh, vh):
    full = pl.BlockSpec((S, D), lambda i: (0, 0))
    return pl.pallas_call(
        _attn0_body,
        grid=(1,),
        in_specs=[pl.BlockSpec((8, D), lambda i: (0, 0)),
                  full, full],
        out_specs=pl.BlockSpec((8, D), lambda i: (0, 0)),
        out_shape=jax.ShapeDtypeStruct((8, D), f32),
    )(qk, knh, vh)


def _tail_body(o_ref, y1r_ref, y2r_ref, woh_ref, wol_ref, bo_ref,
               g_ref, b_ref, w1h_ref, w1l_ref, b1_ref, w2h_ref, w2l_ref,
               b2_ref, wc_ref, bc_ref, out_ref):
    y1f = y1r_ref[...] + _mm3(o_ref[...], woh_ref[...], wol_ref[...]) \
        + bo_ref[...]
    a = _ln(y1f, g_ref[...], b_ref[...])
    hdd = _gelu(_mm3(a, w1h_ref[...], w1l_ref[...]) + b1_ref[...])
    y2f = y2r_ref[...] + _mm3(hdd, w2h_ref[...], w2l_ref[...]) + b2_ref[...]
    hsum = y1f + y2f
    out_ref[...] = jnp.dot(hsum, wc_ref[...], preferred_element_type=f32,
                           precision=HI) + bc_ref[...]


def _tail(o8, y1r, y2r, woh, wol, bo, g, b, w1h, w1l, b1, w2h, w2l, b2,
          wc, bc):
    full = lambda shape: pl.BlockSpec(shape, lambda i: tuple(0 for _ in shape))
    return pl.pallas_call(
        _tail_body,
        grid=(1,),
        in_specs=[
            full((8, D)), full((8, D)), full((8, D)),
            full((D, D)), full((D, D)), full((1, D)), full((1, D)),
            full((1, D)),
            full((D, F)), full((D, F)), full((1, F)),
            full((F, D)), full((F, D)), full((1, D)),
            full((D, 2)), full((1, 2)),
        ],
        out_specs=full((8, 2)),
        out_shape=jax.ShapeDtypeStruct((8, 2), f32),
    )(o8, y1r, y2r, woh, wol, bo, g, b, w1h, w1l, b1, w2h, w2l, b2, wc, bc)


def _wsplit(w):
    hi32 = _rne_bf16(w)
    hi = hi32.astype(bf16)
    lo = _rne_bf16(w - hi32).astype(bf16)
    return hi, lo


def kernel(input_ids, token_emb, pos_emb, ln1_g, ln1_b, Wqk, Wv, Wo, bo,
           ln2_g, ln2_b, W1, b1, W2, b2, Wc, bc):
    ids = input_ids.reshape(S).astype(jnp.int32)
    emb = _gather_sc(token_emb, ids)

    r1 = lambda a: a.reshape(1, -1)
    wqk = [_wsplit(Wqk[i]) for i in range(2)]
    wv = [_wsplit(Wv[i]) for i in range(2)]
    wo = [_wsplit(Wo[i]) for i in range(2)]
    w1 = [_wsplit(W1[i]) for i in range(2)]
    w2 = [_wsplit(W2[i]) for i in range(2)]

    qk0, knh0, vh0, x0 = _ln_qkv(
        emb, pos_emb, r1(ln1_g[0]), r1(ln1_b[0]), *wqk[0], *wv[0])
    o0 = _attn_full(qk0, knh0, vh0)
    y1, a2 = _oproj_ln(o0, x0, *wo[0], r1(bo[0]), r1(ln2_g[0]),
                       r1(ln2_b[0]))
    y2 = _ffn(a2, x0, *w1[0], r1(b1[0]), *w2[0], r1(b2[0]))

    qk1, knh1, vh1 = _ln_qkv(y2, None, r1(ln1_g[1]),
                             r1(ln1_b[1]), *wqk[1], *wv[1])
    o8 = _attn_row0(qk1, knh1, vh1)
    out8 = _tail(o8, lax.slice(y1, (0, 0), (8, D)),
                 lax.slice(y2, (0, 0), (8, D)),
                 *wo[1], r1(bo[1]), r1(ln2_g[1]), r1(ln2_b[1]),
                 *w1[1], r1(b1[1]), *w2[1], r1(b2[1]), Wc, r1(bc))
    return out8[0:1, :]
